# baseline (speedup 1.0000x reference)
import functools

import jax
import jax.numpy as jnp
from jax import lax
from jax.experimental import pallas as pl
from jax.experimental.pallas import tpu as pltpu
from jax.experimental.pallas import tpu_sc as plsc

N_V = 10000
N_C = 40000
E = 160000
D = 128
PAD = 64
ROUNDS = 4

NVP = 10240
NCP = 40960

NCORES = 2
NSUB = 16
NW = NCORES * NSUB
K = 128
GPT = E // NW
W = (GPT + K - 1) // K
SPT = W * K
NSTR = NW * SPT
EPAD = E + 2 * K
NOMATCH = 1 << 30


def _softplus(x):
    return jnp.maximum(x, 0.0) + jnp.log1p(jnp.exp(-jnp.abs(x)))


def _q_body(v_ref, n_ref, w0a, w0b, b0, w1, b1, w2, b2, tp_ref, tn_ref):
    x1 = v_ref[...]
    h = jnp.dot(x1, w0a[...], preferred_element_type=jnp.float32)
    h += jnp.dot(n_ref[...], w0b[...], preferred_element_type=jnp.float32)
    h = jax.nn.relu(h + b0[...])
    h = jax.nn.relu(jnp.dot(h, w1[...], preferred_element_type=jnp.float32) + b1[...])
    q = jnp.dot(h, w2[...], preferred_element_type=jnp.float32) + b2[...]
    tp_ref[...] = jnp.concatenate([_softplus(q), x1[:, :D]], axis=1)
    tn_ref[...] = jnp.concatenate([_softplus(-q), x1[:, D:]], axis=1)


def _c_body(ap_ref, an_ref, c_ref, w0a, w0b, w0c, b0, w1, b1, w2, b2, out_ref):
    ap = ap_ref[...]
    an = an_ref[...]
    e = jnp.exp(-(ap[:, :D] + an[:, :D]))
    l2c = ap[:, D:] + an[:, D:]
    h = jnp.dot(l2c, w0a[...], preferred_element_type=jnp.float32)
    h += jnp.dot(c_ref[...], w0b[...], preferred_element_type=jnp.float32)
    h += jnp.dot(e, w0c[...], preferred_element_type=jnp.float32)
    h = jax.nn.relu(h + b0[...])
    h = jax.nn.relu(jnp.dot(h, w1[...], preferred_element_type=jnp.float32) + b1[...])
    out_ref[...] = jnp.dot(h, w2[...], preferred_element_type=jnp.float32) + b2[...]


def _v_body(v_ref, cp_ref, cn_ref, w0a, w0b, w0c, b0, w1, b1, w2, b2, out_ref):
    h = jnp.dot(v_ref[...], w0a[...], preferred_element_type=jnp.float32)
    h += jnp.dot(cp_ref[...], w0b[...], preferred_element_type=jnp.float32)
    h += jnp.dot(cn_ref[...], w0c[...], preferred_element_type=jnp.float32)
    h = jax.nn.relu(h + b0[...])
    h = jax.nn.relu(jnp.dot(h, w1[...], preferred_element_type=jnp.float32) + b1[...])
    out_ref[...] = jnp.dot(h, w2[...], preferred_element_type=jnp.float32) + b2[...]


def _row_spec(rows, cols):
    return pl.BlockSpec((rows, cols), lambda i: (i, 0))


def _full_spec(shape):
    return pl.BlockSpec(shape, lambda i: tuple(0 for _ in shape))


def _tc_call(body, n_rows, block_rows, in_shapes, out_shapes, blocked_in):
    grid = (n_rows // block_rows,)
    in_specs = []
    for shp, blocked in zip(in_shapes, blocked_in):
        if blocked:
            in_specs.append(_row_spec(block_rows, shp[1]))
        else:
            in_specs.append(_full_spec(shp))
    out_specs = [_row_spec(block_rows, s[1]) for s in out_shapes]
    out_shape = [jax.ShapeDtypeStruct(s, jnp.float32) for s in out_shapes]
    return pl.pallas_call(
        body,
        grid=grid,
        in_specs=in_specs,
        out_specs=out_specs if len(out_specs) > 1 else out_specs[0],
        out_shape=out_shape if len(out_shape) > 1 else out_shape[0],
    )


def _make_gather(cols):
    mesh = plsc.VectorSubcoreMesh(core_axis_name="c", subcore_axis_name="s",
                                  num_cores=NCORES, num_subcores=NSUB)

    @functools.partial(
        pl.kernel,
        mesh=mesh,
        compiler_params=pltpu.CompilerParams(needs_layout_passes=False),
        out_type=jax.ShapeDtypeStruct((NSTR, cols), jnp.float32),
        scratch_types=[
            pltpu.VMEM((K,), jnp.int32),
            pltpu.VMEM((K, cols), jnp.float32),
            pltpu.SemaphoreType.DMA,
        ],
    )
    def gather(tab, gidx, out, gv, rows_v, sem):
        core = lax.axis_index("c")
        sid = lax.axis_index("s")
        wid = sid * NCORES + core
        e0 = wid * GPT
        p0 = wid * SPT

        def step(t, _):
            base = pl.multiple_of(e0 + K * t, 8)
            pltpu.sync_copy(gidx.at[pl.ds(base, K)], gv)
            pltpu.async_copy(tab.at[gv], rows_v, sem).wait()
            pltpu.sync_copy(rows_v, out.at[pl.ds(p0 + K * t, K)])
            return 0
        lax.fori_loop(0, W, step, 0)

    return gather


def _make_segsum_tc(cols, nb_out, cmax):
    def body(gblk_ref, oblk_ref, dst_ref, g_ref, out_ref):
        pid = pl.program_id(0)
        b = oblk_ref[pid]
        prev = oblk_ref[jnp.maximum(pid - 1, 0)]
        dst = dst_ref[0, 0, :]
        local = dst - b * 128
        rows = lax.broadcasted_iota(jnp.int32, (128, K), 0)
        oh = (rows == local[None, :]).astype(jnp.float32)
        partial = jnp.dot(oh, g_ref[...], preferred_element_type=jnp.float32)

        @pl.when((pid == 0) | (prev != b))
        def _():
            out_ref[...] = partial

        @pl.when((pid > 0) & (prev == b))
        def _():
            out_ref[...] += partial

    return pl.pallas_call(
        body,
        grid_spec=pltpu.PrefetchScalarGridSpec(
            num_scalar_prefetch=2,
            grid=(cmax,),
            in_specs=[
                pl.BlockSpec((1, 1, K), lambda i, g, o: (g[i], 0, 0)),
                pl.BlockSpec((K, cols), lambda i, g, o: (g[i], 0)),
            ],
            out_specs=pl.BlockSpec((128, cols), lambda i, g, o: (o[i], 0)),
        ),
        out_shape=jax.ShapeDtypeStruct((nb_out * 128, cols), jnp.float32),
    )


def _edge_plan(scatter_idx, gather_idx, nb_out, cmax):
    perm = jnp.argsort(scatter_idx)
    sdst = scatter_idx[perm].astype(jnp.int32)
    sgid = gather_idx[perm].astype(jnp.int32)
    gpad = jnp.concatenate([sgid, jnp.zeros((EPAD - E,), jnp.int32)])

    e = jnp.arange(E, dtype=jnp.int32)
    pos = (e // GPT) * SPT + e % GPT
    dstr = jnp.full((NSTR,), NOMATCH, jnp.int32).at[pos].set(sdst)

    bs_e = jnp.searchsorted(
        sdst, jnp.arange(nb_out + 1, dtype=jnp.int32) * 128).astype(jnp.int32)
    bs_p = (bs_e // GPT) * SPT + bs_e % GPT
    cs = bs_p[:-1] & ~127
    cnt = jnp.maximum((bs_p[1:] - cs + 127) // 128, 1)
    off = jnp.concatenate(
        [jnp.zeros((1,), jnp.int32), jnp.cumsum(cnt).astype(jnp.int32)])
    total = off[-1]

    i = jnp.arange(cmax, dtype=jnp.int32)
    b = jnp.clip(jnp.searchsorted(off, i, side="right") - 1, 0, nb_out - 1)
    t = i - off[b]
    gblk = jnp.where(i < total, cs[b] // 128 + t, 0)
    oblk = jnp.where(i < total, b, nb_out - 1)
    return gpad, dstr.reshape(NSTR // 128, 1, 128), gblk, oblk


def kernel(l_pos_emb, l_neg_emb, c_emb, pos_src, pos_dst, neg_src, neg_dst,
           q_W0, q_b0, q_W1, q_b1, q_W2, q_b2,
           v_W0, v_b0, v_W1, v_b1, v_W2, v_b2,
           c_W0, c_b0, c_W1, c_b1, c_W2, c_b2):
    f32 = jnp.float32
    NB_C = NCP // 128
    NB_V = NVP // 128
    CMAX_C = NSTR // 128 + 2 * NB_C + 8
    CMAX_V = NSTR // 128 + 2 * NB_V + 8

    pg, pdstr, pgb, pob = _edge_plan(pos_dst, pos_src, NB_C, CMAX_C)
    ng, ndstr, ngb, nob = _edge_plan(neg_dst, neg_src, NB_C, CMAX_C)
    pg2, pdstr2, pgb2, pob2 = _edge_plan(pos_src, pos_dst, NB_V, CMAX_V)
    ng2, ndstr2, ngb2, nob2 = _edge_plan(neg_src, neg_dst, NB_V, CMAX_V)

    v_emb = jnp.zeros((NVP, 2 * D), f32)
    v_emb = v_emb.at[:N_V, :D].set(l_pos_emb).at[:N_V, D:].set(l_neg_emb)
    c = jnp.zeros((NCP, D), f32).at[:N_C].set(c_emb)

    nkey = jax.random.key(42)
    noises = [jax.random.normal(jax.random.fold_in(nkey, r), (N_V, PAD), f32)
              for r in range(ROUNDS)]
    noises = [jnp.zeros((NVP, PAD), f32).at[:N_V].set(nz) for nz in noises]

    q_W0a, q_W0b = q_W0[:2 * D], q_W0[2 * D:]
    v_W0a, v_W0b, v_W0c = v_W0[:2 * D], v_W0[2 * D:3 * D], v_W0[3 * D:]
    c_W0a, c_W0b, c_W0c = c_W0[:D], c_W0[D:2 * D], c_W0[2 * D:]
    qb0, qb1, qb2 = q_b0[None], q_b1[None], q_b2[None]
    vb0, vb1, vb2 = v_b0[None], v_b1[None], v_b2[None]
    cb0, cb1, cb2 = c_b0[None], c_b1[None], c_b2[None]

    BQ = 1024
    q_call = _tc_call(
        _q_body, NVP, BQ,
        [(NVP, 2 * D), (NVP, PAD), q_W0a.shape, q_W0b.shape, qb0.shape,
         q_W1.shape, qb1.shape, q_W2.shape, qb2.shape],
        [(NVP, 2 * D), (NVP, 2 * D)],
        [True, True, False, False, False, False, False, False, False])
    BC = 2048
    c_call = _tc_call(
        _c_body, NCP, BC,
        [(NCP, 2 * D), (NCP, 2 * D), (NCP, D), c_W0a.shape, c_W0b.shape,
         c_W0c.shape, cb0.shape, c_W1.shape, cb1.shape, c_W2.shape, cb2.shape],
        [(NCP, D)],
        [True, True, True] + [False] * 8)
    v_call = _tc_call(
        _v_body, NVP, BQ,
        [(NVP, 2 * D), (NVP, D), (NVP, D), v_W0a.shape, v_W0b.shape,
         v_W0c.shape, vb0.shape, v_W1.shape, vb1.shape, v_W2.shape, vb2.shape],
        [(NVP, 2 * D)],
        [True, True, True] + [False] * 8)

    gather256 = _make_gather(2 * D)
    gather128 = _make_gather(D)
    seg_c = _make_segsum_tc(2 * D, NB_C, CMAX_C)
    seg_v = _make_segsum_tc(D, NB_V, CMAX_V)

    for r in range(ROUNDS):
        tp, tn = q_call(v_emb, noises[r], q_W0a, q_W0b, qb0,
                        q_W1, qb1, q_W2, qb2)
        gp = gather256(tp, pg)
        gn = gather256(tn, ng)
        accp = seg_c(pgb, pob, pdstr, gp)
        accn = seg_c(ngb, nob, ndstr, gn)
        gcp = gather128(c, pg2)
        gcn = gather128(c, ng2)
        cp = seg_v(pgb2, pob2, pdstr2, gcp)
        cn = seg_v(ngb2, nob2, ndstr2, gcn)
        c = c_call(accp, accn, c, c_W0a, c_W0b, c_W0c, cb0,
                   c_W1, cb1, c_W2, cb2)
        v_emb = v_call(v_emb, cp, cn, v_W0a, v_W0b, v_W0c,
                       vb0, v_W1, vb1, v_W2, vb2)

    return v_emb[:N_V, :D], v_emb[:N_V, D:], c[:N_C]


# speedup vs baseline: 1.8331x; 1.8331x over previous
import functools

import jax
import jax.numpy as jnp
from jax import lax
from jax.experimental import pallas as pl
from jax.experimental.pallas import tpu as pltpu
from jax.experimental.pallas import tpu_sc as plsc

N_V = 10000
N_C = 40000
E = 160000
D = 128
PAD = 64
ROUNDS = 4

NVP = 10240
NCP = 40960

NCORES = 2
NSUB = 16
NW = NCORES * NSUB
K = 128
GPT = E // NW
W = (GPT + K - 1) // K
SPT = W * K
NSTR = NW * SPT
EPAD = E + 2 * K
NOMATCH = 1 << 30


def _softplus(x):
    return jnp.maximum(x, 0.0) + jnp.log1p(jnp.exp(-jnp.abs(x)))


def _q_body(v_ref, n_ref, w0a, w0b, b0, w1, b1, w2, b2, tp_ref, tn_ref):
    x1 = v_ref[...]
    h = jnp.dot(x1, w0a[...], preferred_element_type=jnp.float32)
    h += jnp.dot(n_ref[...], w0b[...], preferred_element_type=jnp.float32)
    h = jax.nn.relu(h + b0[...])
    h = jax.nn.relu(jnp.dot(h, w1[...], preferred_element_type=jnp.float32) + b1[...])
    q = jnp.dot(h, w2[...], preferred_element_type=jnp.float32) + b2[...]
    tp_ref[...] = jnp.concatenate([_softplus(q), x1[:, :D]], axis=1)
    tn_ref[...] = jnp.concatenate([_softplus(-q), x1[:, D:]], axis=1)


def _c_body(ap_ref, an_ref, c_ref, w0a, w0b, w0c, b0, w1, b1, w2, b2, out_ref):
    ap = ap_ref[...]
    an = an_ref[...]
    e = jnp.exp(-(ap[:, :D] + an[:, :D]))
    l2c = ap[:, D:] + an[:, D:]
    h = jnp.dot(l2c, w0a[...], preferred_element_type=jnp.float32)
    h += jnp.dot(c_ref[...], w0b[...], preferred_element_type=jnp.float32)
    h += jnp.dot(e, w0c[...], preferred_element_type=jnp.float32)
    h = jax.nn.relu(h + b0[...])
    h = jax.nn.relu(jnp.dot(h, w1[...], preferred_element_type=jnp.float32) + b1[...])
    out_ref[...] = jnp.dot(h, w2[...], preferred_element_type=jnp.float32) + b2[...]


def _v_body(v_ref, cp_ref, cn_ref, w0a, w0b, w0c, b0, w1, b1, w2, b2, out_ref):
    h = jnp.dot(v_ref[...], w0a[...], preferred_element_type=jnp.float32)
    h += jnp.dot(cp_ref[...], w0b[...], preferred_element_type=jnp.float32)
    h += jnp.dot(cn_ref[...], w0c[...], preferred_element_type=jnp.float32)
    h = jax.nn.relu(h + b0[...])
    h = jax.nn.relu(jnp.dot(h, w1[...], preferred_element_type=jnp.float32) + b1[...])
    out_ref[...] = jnp.dot(h, w2[...], preferred_element_type=jnp.float32) + b2[...]


def _row_spec(rows, cols):
    return pl.BlockSpec((rows, cols), lambda i: (i, 0))


def _full_spec(shape):
    return pl.BlockSpec(shape, lambda i: tuple(0 for _ in shape))


def _tc_call(body, n_rows, block_rows, in_shapes, out_shapes, blocked_in):
    grid = (n_rows // block_rows,)
    in_specs = []
    for shp, blocked in zip(in_shapes, blocked_in):
        if blocked:
            in_specs.append(_row_spec(block_rows, shp[1]))
        else:
            in_specs.append(_full_spec(shp))
    out_specs = [_row_spec(block_rows, s[1]) for s in out_shapes]
    out_shape = [jax.ShapeDtypeStruct(s, jnp.float32) for s in out_shapes]
    return pl.pallas_call(
        body,
        grid=grid,
        in_specs=in_specs,
        out_specs=out_specs if len(out_specs) > 1 else out_specs[0],
        out_shape=out_shape if len(out_shape) > 1 else out_shape[0],
    )


def _make_gather(cols):
    mesh = plsc.VectorSubcoreMesh(core_axis_name="c", subcore_axis_name="s",
                                  num_cores=NCORES, num_subcores=NSUB)

    @functools.partial(
        pl.kernel,
        mesh=mesh,
        compiler_params=pltpu.CompilerParams(needs_layout_passes=False),
        out_type=jax.ShapeDtypeStruct((NSTR, cols), jnp.float32),
        scratch_types=[
            pltpu.VMEM((SPT,), jnp.int32),
            pltpu.VMEM((K, cols), jnp.float32),
            pltpu.VMEM((K, cols), jnp.float32),
            pltpu.SemaphoreType.DMA,
            pltpu.SemaphoreType.DMA,
            pltpu.SemaphoreType.DMA,
        ],
    )
    def gather(tab, gidx, out, iv, r0, r1, semg, semw0, semw1):
        core = lax.axis_index("c")
        sid = lax.axis_index("s")
        wid = sid * NCORES + core
        e0 = pl.multiple_of(wid * GPT, 8)
        p0 = wid * SPT

        pltpu.sync_copy(gidx.at[pl.ds(e0, SPT)], iv)

        bufs = ((r0, semw0), (r1, semw1))

        def step(u, _):
            for h in range(2):
                t = 2 * u + h
                rb, semw = bufs[h]
                @pl.when(t >= 2)
                def _():
                    pltpu.make_async_copy(
                        rb, out.at[pl.ds(p0 + K * (t - 2), K)], semw).wait()
                pltpu.async_copy(
                    tab.at[iv.at[pl.ds(pl.multiple_of(K * t, 8), K)]],
                    rb, semg).wait()
                pltpu.async_copy(rb, out.at[pl.ds(p0 + K * t, K)], semw)
            return 0
        lax.fori_loop(0, W // 2, step, 0)

        for h in range(2):
            rb, semw = bufs[h]
            t = W - 2 + h
            pltpu.make_async_copy(
                rb, out.at[pl.ds(p0 + K * t, K)], semw).wait()

    return gather


OB = 256
CH = 512


def _make_segsum_tc(cols, nb_out, cmax):
    def body(gblk_ref, oblk_ref, dst_ref, g_ref, out_ref):
        pid = pl.program_id(0)
        b = oblk_ref[pid]
        prev = oblk_ref[jnp.maximum(pid - 1, 0)]
        dst = dst_ref[0, 0, :]
        local = dst - b * OB
        rows = lax.broadcasted_iota(jnp.int32, (OB, CH), 0)
        oh = (rows == local[None, :]).astype(jnp.float32)
        partial = jnp.dot(oh, g_ref[...], preferred_element_type=jnp.float32)

        @pl.when((pid == 0) | (prev != b))
        def _():
            out_ref[...] = partial

        @pl.when((pid > 0) & (prev == b))
        def _():
            out_ref[...] += partial

    return pl.pallas_call(
        body,
        grid_spec=pltpu.PrefetchScalarGridSpec(
            num_scalar_prefetch=2,
            grid=(cmax,),
            in_specs=[
                pl.BlockSpec((1, 1, CH), lambda i, g, o: (g[i], 0, 0)),
                pl.BlockSpec((CH, cols), lambda i, g, o: (g[i], 0)),
            ],
            out_specs=pl.BlockSpec((OB, cols), lambda i, g, o: (o[i], 0)),
        ),
        out_shape=jax.ShapeDtypeStruct((nb_out * OB, cols), jnp.float32),
    )


def _edge_plan(scatter_idx, gather_idx, nb_out, cmax):
    perm = jnp.argsort(scatter_idx)
    sdst = scatter_idx[perm].astype(jnp.int32)
    sgid = gather_idx[perm].astype(jnp.int32)
    gpad = jnp.concatenate([sgid, jnp.zeros((EPAD - E,), jnp.int32)])

    e = jnp.arange(E, dtype=jnp.int32)
    pos = (e // GPT) * SPT + e % GPT
    dstr = jnp.full((NSTR,), NOMATCH, jnp.int32).at[pos].set(sdst)

    bs_e = jnp.searchsorted(
        sdst, jnp.arange(nb_out + 1, dtype=jnp.int32) * OB).astype(jnp.int32)
    bs_p = (bs_e // GPT) * SPT + bs_e % GPT
    cs = bs_p[:-1] & ~(CH - 1)
    cnt = jnp.maximum((bs_p[1:] - cs + CH - 1) // CH, 1)
    off = jnp.concatenate(
        [jnp.zeros((1,), jnp.int32), jnp.cumsum(cnt).astype(jnp.int32)])
    total = off[-1]

    i = jnp.arange(cmax, dtype=jnp.int32)
    b = jnp.clip(jnp.searchsorted(off, i, side="right") - 1, 0, nb_out - 1)
    t = i - off[b]
    gblk = jnp.where(i < total, cs[b] // CH + t, 0)
    oblk = jnp.where(i < total, b, nb_out - 1)
    return gpad, dstr.reshape(NSTR // CH, 1, CH), gblk, oblk


def kernel(l_pos_emb, l_neg_emb, c_emb, pos_src, pos_dst, neg_src, neg_dst,
           q_W0, q_b0, q_W1, q_b1, q_W2, q_b2,
           v_W0, v_b0, v_W1, v_b1, v_W2, v_b2,
           c_W0, c_b0, c_W1, c_b1, c_W2, c_b2):
    f32 = jnp.float32
    NB_C = NCP // OB
    NB_V = NVP // OB
    CMAX_C = NSTR // CH + 2 * NB_C + 8
    CMAX_V = NSTR // CH + 2 * NB_V + 8

    pg, pdstr, pgb, pob = _edge_plan(pos_dst, pos_src, NB_C, CMAX_C)
    ng, ndstr, ngb, nob = _edge_plan(neg_dst, neg_src, NB_C, CMAX_C)
    pg2, pdstr2, pgb2, pob2 = _edge_plan(pos_src, pos_dst, NB_V, CMAX_V)
    ng2, ndstr2, ngb2, nob2 = _edge_plan(neg_src, neg_dst, NB_V, CMAX_V)

    v_emb = jnp.zeros((NVP, 2 * D), f32)
    v_emb = v_emb.at[:N_V, :D].set(l_pos_emb).at[:N_V, D:].set(l_neg_emb)
    c = jnp.zeros((NCP, D), f32).at[:N_C].set(c_emb)

    nkey = jax.random.key(42)
    noises = [jax.random.normal(jax.random.fold_in(nkey, r), (N_V, PAD), f32)
              for r in range(ROUNDS)]
    noises = [jnp.zeros((NVP, PAD), f32).at[:N_V].set(nz) for nz in noises]

    q_W0a, q_W0b = q_W0[:2 * D], q_W0[2 * D:]
    v_W0a, v_W0b, v_W0c = v_W0[:2 * D], v_W0[2 * D:3 * D], v_W0[3 * D:]
    c_W0a, c_W0b, c_W0c = c_W0[:D], c_W0[D:2 * D], c_W0[2 * D:]
    qb0, qb1, qb2 = q_b0[None], q_b1[None], q_b2[None]
    vb0, vb1, vb2 = v_b0[None], v_b1[None], v_b2[None]
    cb0, cb1, cb2 = c_b0[None], c_b1[None], c_b2[None]

    BQ = 1024
    q_call = _tc_call(
        _q_body, NVP, BQ,
        [(NVP, 2 * D), (NVP, PAD), q_W0a.shape, q_W0b.shape, qb0.shape,
         q_W1.shape, qb1.shape, q_W2.shape, qb2.shape],
        [(NVP, 2 * D), (NVP, 2 * D)],
        [True, True, False, False, False, False, False, False, False])
    BC = 2048
    c_call = _tc_call(
        _c_body, NCP, BC,
        [(NCP, 2 * D), (NCP, 2 * D), (NCP, D), c_W0a.shape, c_W0b.shape,
         c_W0c.shape, cb0.shape, c_W1.shape, cb1.shape, c_W2.shape, cb2.shape],
        [(NCP, D)],
        [True, True, True] + [False] * 8)
    v_call = _tc_call(
        _v_body, NVP, BQ,
        [(NVP, 2 * D), (NVP, D), (NVP, D), v_W0a.shape, v_W0b.shape,
         v_W0c.shape, vb0.shape, v_W1.shape, vb1.shape, v_W2.shape, vb2.shape],
        [(NVP, 2 * D)],
        [True, True, True] + [False] * 8)

    gather256 = _make_gather(2 * D)
    gather128 = _make_gather(D)
    seg_c = _make_segsum_tc(2 * D, NB_C, CMAX_C)
    seg_v = _make_segsum_tc(D, NB_V, CMAX_V)

    for r in range(ROUNDS):
        tp, tn = q_call(v_emb, noises[r], q_W0a, q_W0b, qb0,
                        q_W1, qb1, q_W2, qb2)
        gp = gather256(tp, pg)
        gn = gather256(tn, ng)
        accp = seg_c(pgb, pob, pdstr, gp)
        accn = seg_c(ngb, nob, ndstr, gn)
        gcp = gather128(c, pg2)
        gcn = gather128(c, ng2)
        cp = seg_v(pgb2, pob2, pdstr2, gcp)
        cn = seg_v(ngb2, nob2, ndstr2, gcn)
        c = c_call(accp, accn, c, c_W0a, c_W0b, c_W0c, cb0,
                   c_W1, cb1, c_W2, cb2)
        v_emb = v_call(v_emb, cp, cn, v_W0a, v_W0b, v_W0c,
                       vb0, v_W1, vb1, v_W2, vb2)

    return v_emb[:N_V, :D], v_emb[:N_V, D:], c[:N_C]


# speedup vs baseline: 2.2816x; 1.2446x over previous
import functools

import jax
import jax.numpy as jnp
from jax import lax
from jax.experimental import pallas as pl
from jax.experimental.pallas import tpu as pltpu
from jax.experimental.pallas import tpu_sc as plsc

N_V = 10000
N_C = 40000
E = 160000
D = 128
PAD = 64
ROUNDS = 4

NVP = 10240
NCP = 40960

NCORES = 2
NSUB = 16
NW = NCORES * NSUB
K = 128
GPT = E // NW
W = (GPT + K - 1) // K
SPT = W * K
NSTR = NW * SPT
EPAD = E + 2 * K
NOMATCH = 1 << 30


def _softplus(x):
    return jnp.maximum(x, 0.0) + jnp.log1p(jnp.exp(-jnp.abs(x)))


def _q_body(v_ref, n_ref, w0a, w0b, b0, w1, b1, w2, b2, tp_ref, tn_ref):
    x1 = v_ref[...]
    h = jnp.dot(x1, w0a[...], preferred_element_type=jnp.float32)
    h += jnp.dot(n_ref[...], w0b[...], preferred_element_type=jnp.float32)
    h = jax.nn.relu(h + b0[...])
    h = jax.nn.relu(jnp.dot(h, w1[...], preferred_element_type=jnp.float32) + b1[...])
    q = jnp.dot(h, w2[...], preferred_element_type=jnp.float32) + b2[...]
    tp_ref[...] = jnp.concatenate([_softplus(q), x1[:, :D]], axis=1)
    tn_ref[...] = jnp.concatenate([_softplus(-q), x1[:, D:]], axis=1)


def _c_body(ap_ref, an_ref, c_ref, w0a, w0b, w0c, b0, w1, b1, w2, b2, out_ref):
    ap = ap_ref[...]
    an = an_ref[...]
    e = jnp.exp(-(ap[:, :D] + an[:, :D]))
    l2c = ap[:, D:] + an[:, D:]
    h = jnp.dot(l2c, w0a[...], preferred_element_type=jnp.float32)
    h += jnp.dot(c_ref[...], w0b[...], preferred_element_type=jnp.float32)
    h += jnp.dot(e, w0c[...], preferred_element_type=jnp.float32)
    h = jax.nn.relu(h + b0[...])
    h = jax.nn.relu(jnp.dot(h, w1[...], preferred_element_type=jnp.float32) + b1[...])
    out_ref[...] = jnp.dot(h, w2[...], preferred_element_type=jnp.float32) + b2[...]


def _v_body(v_ref, cp_ref, cn_ref, w0a, w0b, w0c, b0, w1, b1, w2, b2, out_ref):
    h = jnp.dot(v_ref[...], w0a[...], preferred_element_type=jnp.float32)
    h += jnp.dot(cp_ref[...], w0b[...], preferred_element_type=jnp.float32)
    h += jnp.dot(cn_ref[...], w0c[...], preferred_element_type=jnp.float32)
    h = jax.nn.relu(h + b0[...])
    h = jax.nn.relu(jnp.dot(h, w1[...], preferred_element_type=jnp.float32) + b1[...])
    out_ref[...] = jnp.dot(h, w2[...], preferred_element_type=jnp.float32) + b2[...]


def _row_spec(rows, cols):
    return pl.BlockSpec((rows, cols), lambda i: (i, 0))


def _full_spec(shape):
    return pl.BlockSpec(shape, lambda i: tuple(0 for _ in shape))


def _tc_call(body, n_rows, block_rows, in_shapes, out_shapes, blocked_in):
    grid = (n_rows // block_rows,)
    in_specs = []
    for shp, blocked in zip(in_shapes, blocked_in):
        if blocked:
            in_specs.append(_row_spec(block_rows, shp[1]))
        else:
            in_specs.append(_full_spec(shp))
    out_specs = [_row_spec(block_rows, s[1]) for s in out_shapes]
    out_shape = [jax.ShapeDtypeStruct(s, jnp.float32) for s in out_shapes]
    return pl.pallas_call(
        body,
        grid=grid,
        in_specs=in_specs,
        out_specs=out_specs if len(out_specs) > 1 else out_specs[0],
        out_shape=out_shape if len(out_shape) > 1 else out_shape[0],
    )


def _make_gather(cols):
    mesh = plsc.VectorSubcoreMesh(core_axis_name="c", subcore_axis_name="s",
                                  num_cores=NCORES, num_subcores=NSUB)

    @functools.partial(
        pl.kernel,
        mesh=mesh,
        compiler_params=pltpu.CompilerParams(needs_layout_passes=False),
        out_type=jax.ShapeDtypeStruct((NSTR, cols), jnp.float32),
        scratch_types=[
            pltpu.VMEM((SPT,), jnp.int32),
            pltpu.VMEM((K, cols), jnp.float32),
            pltpu.VMEM((K, cols), jnp.float32),
            pltpu.SemaphoreType.DMA,
            pltpu.SemaphoreType.DMA,
            pltpu.SemaphoreType.DMA,
        ],
    )
    def gather(tab, gidx, out, iv, r0, r1, semg, semw0, semw1):
        core = lax.axis_index("c")
        sid = lax.axis_index("s")
        wid = sid * NCORES + core
        e0 = pl.multiple_of(wid * GPT, 8)
        p0 = wid * SPT

        pltpu.sync_copy(gidx.at[pl.ds(e0, SPT)], iv)

        bufs = ((r0, semw0), (r1, semw1))

        def step(u, _):
            for h in range(2):
                t = 2 * u + h
                rb, semw = bufs[h]
                @pl.when(t >= 2)
                def _():
                    pltpu.make_async_copy(
                        rb, out.at[pl.ds(p0 + K * (t - 2), K)], semw).wait()
                pltpu.async_copy(
                    tab.at[iv.at[pl.ds(pl.multiple_of(K * t, 8), K)]],
                    rb, semg).wait()
                pltpu.async_copy(rb, out.at[pl.ds(p0 + K * t, K)], semw)
            return 0
        lax.fori_loop(0, W // 2, step, 0)

        for h in range(2):
            rb, semw = bufs[h]
            t = W - 2 + h
            pltpu.make_async_copy(
                rb, out.at[pl.ds(p0 + K * t, K)], semw).wait()

    return gather


OB = 512
CH = 1024


def _make_segsum_tc(cols, nb_out, cmax):
    def body(gblk_ref, oblk_ref, dst_ref, g_ref, out_ref):
        pid = pl.program_id(0)
        b = oblk_ref[pid]
        prev = oblk_ref[jnp.maximum(pid - 1, 0)]
        dst = dst_ref[0, 0, :]
        local = dst - b * OB
        rows = lax.broadcasted_iota(jnp.int32, (OB, CH), 0)
        oh = (rows == local[None, :]).astype(jnp.float32)
        partial = jnp.dot(oh, g_ref[...], preferred_element_type=jnp.float32)

        @pl.when((pid == 0) | (prev != b))
        def _():
            out_ref[...] = partial

        @pl.when((pid > 0) & (prev == b))
        def _():
            out_ref[...] += partial

    return pl.pallas_call(
        body,
        grid_spec=pltpu.PrefetchScalarGridSpec(
            num_scalar_prefetch=2,
            grid=(cmax,),
            in_specs=[
                pl.BlockSpec((1, 1, CH), lambda i, g, o: (g[i], 0, 0)),
                pl.BlockSpec((CH, cols), lambda i, g, o: (g[i], 0)),
            ],
            out_specs=pl.BlockSpec((OB, cols), lambda i, g, o: (o[i], 0)),
        ),
        out_shape=jax.ShapeDtypeStruct((nb_out * OB, cols), jnp.float32),
    )


def _edge_plan(scatter_idx, gather_idx, nb_out, cmax):
    perm = jnp.argsort(scatter_idx)
    sdst = scatter_idx[perm].astype(jnp.int32)
    sgid = gather_idx[perm].astype(jnp.int32)
    gpad = jnp.concatenate([sgid, jnp.zeros((EPAD - E,), jnp.int32)])

    e = jnp.arange(E, dtype=jnp.int32)
    pos = (e // GPT) * SPT + e % GPT
    dstr = jnp.full((NSTR,), NOMATCH, jnp.int32).at[pos].set(sdst)

    bs_e = jnp.searchsorted(
        sdst, jnp.arange(nb_out + 1, dtype=jnp.int32) * OB).astype(jnp.int32)
    bs_p = (bs_e // GPT) * SPT + bs_e % GPT
    cs = bs_p[:-1] & ~(CH - 1)
    cnt = jnp.maximum((bs_p[1:] - cs + CH - 1) // CH, 1)
    off = jnp.concatenate(
        [jnp.zeros((1,), jnp.int32), jnp.cumsum(cnt).astype(jnp.int32)])
    total = off[-1]

    i = jnp.arange(cmax, dtype=jnp.int32)
    b = jnp.clip(jnp.searchsorted(off, i, side="right") - 1, 0, nb_out - 1)
    t = i - off[b]
    gblk = jnp.where(i < total, cs[b] // CH + t, 0)
    oblk = jnp.where(i < total, b, nb_out - 1)
    return gpad, dstr.reshape(NSTR // CH, 1, CH), gblk, oblk


def kernel(l_pos_emb, l_neg_emb, c_emb, pos_src, pos_dst, neg_src, neg_dst,
           q_W0, q_b0, q_W1, q_b1, q_W2, q_b2,
           v_W0, v_b0, v_W1, v_b1, v_W2, v_b2,
           c_W0, c_b0, c_W1, c_b1, c_W2, c_b2):
    f32 = jnp.float32
    NB_C = NCP // OB
    NB_V = NVP // OB
    CMAX_C = NSTR // CH + 2 * NB_C + 8
    CMAX_V = NSTR // CH + 2 * NB_V + 8

    pg, pdstr, pgb, pob = _edge_plan(pos_dst, pos_src, NB_C, CMAX_C)
    ng, ndstr, ngb, nob = _edge_plan(neg_dst, neg_src, NB_C, CMAX_C)
    pg2, pdstr2, pgb2, pob2 = _edge_plan(pos_src, pos_dst, NB_V, CMAX_V)
    ng2, ndstr2, ngb2, nob2 = _edge_plan(neg_src, neg_dst, NB_V, CMAX_V)

    v_emb = jnp.zeros((NVP, 2 * D), f32)
    v_emb = v_emb.at[:N_V, :D].set(l_pos_emb).at[:N_V, D:].set(l_neg_emb)
    c = jnp.zeros((NCP, D), f32).at[:N_C].set(c_emb)

    nkey = jax.random.key(42)
    noises = [jax.random.normal(jax.random.fold_in(nkey, r), (N_V, PAD), f32)
              for r in range(ROUNDS)]
    noises = [jnp.zeros((NVP, PAD), f32).at[:N_V].set(nz) for nz in noises]

    q_W0a, q_W0b = q_W0[:2 * D], q_W0[2 * D:]
    v_W0a, v_W0b, v_W0c = v_W0[:2 * D], v_W0[2 * D:3 * D], v_W0[3 * D:]
    c_W0a, c_W0b, c_W0c = c_W0[:D], c_W0[D:2 * D], c_W0[2 * D:]
    qb0, qb1, qb2 = q_b0[None], q_b1[None], q_b2[None]
    vb0, vb1, vb2 = v_b0[None], v_b1[None], v_b2[None]
    cb0, cb1, cb2 = c_b0[None], c_b1[None], c_b2[None]

    BQ = 1024
    q_call = _tc_call(
        _q_body, NVP, BQ,
        [(NVP, 2 * D), (NVP, PAD), q_W0a.shape, q_W0b.shape, qb0.shape,
         q_W1.shape, qb1.shape, q_W2.shape, qb2.shape],
        [(NVP, 2 * D), (NVP, 2 * D)],
        [True, True, False, False, False, False, False, False, False])
    BC = 2048
    c_call = _tc_call(
        _c_body, NCP, BC,
        [(NCP, 2 * D), (NCP, 2 * D), (NCP, D), c_W0a.shape, c_W0b.shape,
         c_W0c.shape, cb0.shape, c_W1.shape, cb1.shape, c_W2.shape, cb2.shape],
        [(NCP, D)],
        [True, True, True] + [False] * 8)
    v_call = _tc_call(
        _v_body, NVP, BQ,
        [(NVP, 2 * D), (NVP, D), (NVP, D), v_W0a.shape, v_W0b.shape,
         v_W0c.shape, vb0.shape, v_W1.shape, vb1.shape, v_W2.shape, vb2.shape],
        [(NVP, 2 * D)],
        [True, True, True] + [False] * 8)

    gather256 = _make_gather(2 * D)
    gather128 = _make_gather(D)
    seg_c = _make_segsum_tc(2 * D, NB_C, CMAX_C)
    seg_v = _make_segsum_tc(D, NB_V, CMAX_V)

    for r in range(ROUNDS):
        tp, tn = q_call(v_emb, noises[r], q_W0a, q_W0b, qb0,
                        q_W1, qb1, q_W2, qb2)
        gp = gather256(tp, pg)
        gn = gather256(tn, ng)
        accp = seg_c(pgb, pob, pdstr, gp)
        accn = seg_c(ngb, nob, ndstr, gn)
        gcp = gather128(c, pg2)
        gcn = gather128(c, ng2)
        cp = seg_v(pgb2, pob2, pdstr2, gcp)
        cn = seg_v(ngb2, nob2, ndstr2, gcn)
        c = c_call(accp, accn, c, c_W0a, c_W0b, c_W0c, cb0,
                   c_W1, cb1, c_W2, cb2)
        v_emb = v_call(v_emb, cp, cn, v_W0a, v_W0b, v_W0c,
                       vb0, v_W1, vb1, v_W2, vb2)

    return v_emb[:N_V, :D], v_emb[:N_V, D:], c[:N_C]


# speedup vs baseline: 2.3994x; 1.0516x over previous
import functools

import jax
import jax.numpy as jnp
from jax import lax
from jax.experimental import pallas as pl
from jax.experimental.pallas import tpu as pltpu
from jax.experimental.pallas import tpu_sc as plsc

N_V = 10000
N_C = 40000
E = 160000
D = 128
PAD = 64
ROUNDS = 4

NVP = 10240
NCP = 40960

NCORES = 2
NSUB = 16
NW = NCORES * NSUB
K = 128
GPT = E // NW
W = (GPT + K - 1) // K
SPT = W * K
NSTR = NW * SPT
EPAD = E + 2 * K
NOMATCH = 1 << 30


def _softplus(x):
    return jnp.maximum(x, 0.0) + jnp.log1p(jnp.exp(-jnp.abs(x)))


def _q_body(v_ref, n_ref, w0a, w0b, b0, w1, b1, w2, b2, tp_ref, tn_ref):
    x1 = v_ref[...]
    h = jnp.dot(x1, w0a[...], preferred_element_type=jnp.float32)
    h += jnp.dot(n_ref[...], w0b[...], preferred_element_type=jnp.float32)
    h = jax.nn.relu(h + b0[...])
    h = jax.nn.relu(jnp.dot(h, w1[...], preferred_element_type=jnp.float32) + b1[...])
    q = jnp.dot(h, w2[...], preferred_element_type=jnp.float32) + b2[...]
    tp_ref[...] = jnp.concatenate([_softplus(q), x1[:, :D]], axis=1)
    tn_ref[...] = jnp.concatenate([_softplus(-q), x1[:, D:]], axis=1)


def _c_body(ap_ref, an_ref, c_ref, w0a, w0b, w0c, b0, w1, b1, w2, b2, out_ref):
    ap = ap_ref[...]
    an = an_ref[...]
    e = jnp.exp(-(ap[:, :D] + an[:, :D]))
    l2c = ap[:, D:] + an[:, D:]
    h = jnp.dot(l2c, w0a[...], preferred_element_type=jnp.float32)
    h += jnp.dot(c_ref[...], w0b[...], preferred_element_type=jnp.float32)
    h += jnp.dot(e, w0c[...], preferred_element_type=jnp.float32)
    h = jax.nn.relu(h + b0[...])
    h = jax.nn.relu(jnp.dot(h, w1[...], preferred_element_type=jnp.float32) + b1[...])
    out_ref[...] = jnp.dot(h, w2[...], preferred_element_type=jnp.float32) + b2[...]


def _v_body(v_ref, cp_ref, cn_ref, w0a, w0b, w0c, b0, w1, b1, w2, b2, out_ref):
    h = jnp.dot(v_ref[...], w0a[...], preferred_element_type=jnp.float32)
    h += jnp.dot(cp_ref[...], w0b[...], preferred_element_type=jnp.float32)
    h += jnp.dot(cn_ref[...], w0c[...], preferred_element_type=jnp.float32)
    h = jax.nn.relu(h + b0[...])
    h = jax.nn.relu(jnp.dot(h, w1[...], preferred_element_type=jnp.float32) + b1[...])
    out_ref[...] = jnp.dot(h, w2[...], preferred_element_type=jnp.float32) + b2[...]


def _row_spec(rows, cols):
    return pl.BlockSpec((rows, cols), lambda i: (i, 0))


def _full_spec(shape):
    return pl.BlockSpec(shape, lambda i: tuple(0 for _ in shape))


def _tc_call(body, n_rows, block_rows, in_shapes, out_shapes, blocked_in):
    grid = (n_rows // block_rows,)
    in_specs = []
    for shp, blocked in zip(in_shapes, blocked_in):
        if blocked:
            in_specs.append(_row_spec(block_rows, shp[1]))
        else:
            in_specs.append(_full_spec(shp))
    out_specs = [_row_spec(block_rows, s[1]) for s in out_shapes]
    out_shape = [jax.ShapeDtypeStruct(s, jnp.float32) for s in out_shapes]
    return pl.pallas_call(
        body,
        grid=grid,
        in_specs=in_specs,
        out_specs=out_specs if len(out_specs) > 1 else out_specs[0],
        out_shape=out_shape if len(out_shape) > 1 else out_shape[0],
    )


def _make_gather(cols):
    mesh = plsc.VectorSubcoreMesh(core_axis_name="c", subcore_axis_name="s",
                                  num_cores=NCORES, num_subcores=NSUB)

    @functools.partial(
        pl.kernel,
        mesh=mesh,
        compiler_params=pltpu.CompilerParams(needs_layout_passes=False),
        out_type=jax.ShapeDtypeStruct((NSTR, cols), jnp.float32),
        scratch_types=[
            pltpu.VMEM((SPT,), jnp.int32),
            pltpu.VMEM((K, cols), jnp.float32),
            pltpu.VMEM((K, cols), jnp.float32),
            pltpu.SemaphoreType.DMA,
            pltpu.SemaphoreType.DMA,
            pltpu.SemaphoreType.DMA,
        ],
    )
    def gather(tab, gidx, out, iv, r0, r1, semg, semw0, semw1):
        core = lax.axis_index("c")
        sid = lax.axis_index("s")
        wid = sid * NCORES + core
        e0 = pl.multiple_of(wid * GPT, 8)
        p0 = wid * SPT

        pltpu.sync_copy(gidx.at[pl.ds(e0, SPT)], iv)

        bufs = ((r0, semw0), (r1, semw1))

        def step(u, _):
            for h in range(2):
                t = 2 * u + h
                rb, semw = bufs[h]
                @pl.when(t >= 2)
                def _():
                    pltpu.make_async_copy(
                        rb, out.at[pl.ds(p0 + K * (t - 2), K)], semw).wait()
                pltpu.async_copy(
                    tab.at[iv.at[pl.ds(pl.multiple_of(K * t, 8), K)]],
                    rb, semg).wait()
                pltpu.async_copy(rb, out.at[pl.ds(p0 + K * t, K)], semw)
            return 0
        lax.fori_loop(0, W // 2, step, 0)

        for h in range(2):
            rb, semw = bufs[h]
            t = W - 2 + h
            pltpu.make_async_copy(
                rb, out.at[pl.ds(p0 + K * t, K)], semw).wait()

    return gather


OB = 1024
CH = 2048


def _make_segsum_tc(cols, nb_out, cmax):
    def body(gblk_ref, oblk_ref, dst_ref, g_ref, out_ref):
        pid = pl.program_id(0)
        b = oblk_ref[pid]
        prev = oblk_ref[jnp.maximum(pid - 1, 0)]
        dst = dst_ref[0, 0, :]
        local = dst - b * OB
        rows = lax.broadcasted_iota(jnp.int32, (OB, CH), 0)
        oh = (rows == local[None, :]).astype(jnp.float32)
        partial = jnp.dot(oh, g_ref[...], preferred_element_type=jnp.float32)

        @pl.when((pid == 0) | (prev != b))
        def _():
            out_ref[...] = partial

        @pl.when((pid > 0) & (prev == b))
        def _():
            out_ref[...] += partial

    return pl.pallas_call(
        body,
        grid_spec=pltpu.PrefetchScalarGridSpec(
            num_scalar_prefetch=2,
            grid=(cmax,),
            in_specs=[
                pl.BlockSpec((1, 1, CH), lambda i, g, o: (g[i], 0, 0)),
                pl.BlockSpec((CH, cols), lambda i, g, o: (g[i], 0)),
            ],
            out_specs=pl.BlockSpec((OB, cols), lambda i, g, o: (o[i], 0)),
        ),
        out_shape=jax.ShapeDtypeStruct((nb_out * OB, cols), jnp.float32),
    )


def _edge_plan(scatter_idx, gather_idx, nb_out, cmax):
    perm = jnp.argsort(scatter_idx)
    sdst = scatter_idx[perm].astype(jnp.int32)
    sgid = gather_idx[perm].astype(jnp.int32)
    gpad = jnp.concatenate([sgid, jnp.zeros((EPAD - E,), jnp.int32)])

    e = jnp.arange(E, dtype=jnp.int32)
    pos = (e // GPT) * SPT + e % GPT
    dstr = jnp.full((NSTR,), NOMATCH, jnp.int32).at[pos].set(sdst)

    bs_e = jnp.searchsorted(
        sdst, jnp.arange(nb_out + 1, dtype=jnp.int32) * OB).astype(jnp.int32)
    bs_p = (bs_e // GPT) * SPT + bs_e % GPT
    cs = bs_p[:-1] & ~(CH - 1)
    cnt = jnp.maximum((bs_p[1:] - cs + CH - 1) // CH, 1)
    off = jnp.concatenate(
        [jnp.zeros((1,), jnp.int32), jnp.cumsum(cnt).astype(jnp.int32)])
    total = off[-1]

    i = jnp.arange(cmax, dtype=jnp.int32)
    b = jnp.clip(jnp.searchsorted(off, i, side="right") - 1, 0, nb_out - 1)
    t = i - off[b]
    gblk = jnp.where(i < total, cs[b] // CH + t, 0)
    oblk = jnp.where(i < total, b, nb_out - 1)
    return gpad, dstr.reshape(NSTR // CH, 1, CH), gblk, oblk


def kernel(l_pos_emb, l_neg_emb, c_emb, pos_src, pos_dst, neg_src, neg_dst,
           q_W0, q_b0, q_W1, q_b1, q_W2, q_b2,
           v_W0, v_b0, v_W1, v_b1, v_W2, v_b2,
           c_W0, c_b0, c_W1, c_b1, c_W2, c_b2):
    f32 = jnp.float32
    NB_C = NCP // OB
    NB_V = NVP // OB
    CMAX_C = NSTR // CH + 2 * NB_C + 8
    CMAX_V = NSTR // CH + 2 * NB_V + 8

    pg, pdstr, pgb, pob = _edge_plan(pos_dst, pos_src, NB_C, CMAX_C)
    ng, ndstr, ngb, nob = _edge_plan(neg_dst, neg_src, NB_C, CMAX_C)
    pg2, pdstr2, pgb2, pob2 = _edge_plan(pos_src, pos_dst, NB_V, CMAX_V)
    ng2, ndstr2, ngb2, nob2 = _edge_plan(neg_src, neg_dst, NB_V, CMAX_V)

    v_emb = jnp.zeros((NVP, 2 * D), f32)
    v_emb = v_emb.at[:N_V, :D].set(l_pos_emb).at[:N_V, D:].set(l_neg_emb)
    c = jnp.zeros((NCP, D), f32).at[:N_C].set(c_emb)

    nkey = jax.random.key(42)
    noises = [jax.random.normal(jax.random.fold_in(nkey, r), (N_V, PAD), f32)
              for r in range(ROUNDS)]
    noises = [jnp.zeros((NVP, PAD), f32).at[:N_V].set(nz) for nz in noises]

    q_W0a, q_W0b = q_W0[:2 * D], q_W0[2 * D:]
    v_W0a, v_W0b, v_W0c = v_W0[:2 * D], v_W0[2 * D:3 * D], v_W0[3 * D:]
    c_W0a, c_W0b, c_W0c = c_W0[:D], c_W0[D:2 * D], c_W0[2 * D:]
    qb0, qb1, qb2 = q_b0[None], q_b1[None], q_b2[None]
    vb0, vb1, vb2 = v_b0[None], v_b1[None], v_b2[None]
    cb0, cb1, cb2 = c_b0[None], c_b1[None], c_b2[None]

    BQ = 1024
    q_call = _tc_call(
        _q_body, NVP, BQ,
        [(NVP, 2 * D), (NVP, PAD), q_W0a.shape, q_W0b.shape, qb0.shape,
         q_W1.shape, qb1.shape, q_W2.shape, qb2.shape],
        [(NVP, 2 * D), (NVP, 2 * D)],
        [True, True, False, False, False, False, False, False, False])
    BC = 2048
    c_call = _tc_call(
        _c_body, NCP, BC,
        [(NCP, 2 * D), (NCP, 2 * D), (NCP, D), c_W0a.shape, c_W0b.shape,
         c_W0c.shape, cb0.shape, c_W1.shape, cb1.shape, c_W2.shape, cb2.shape],
        [(NCP, D)],
        [True, True, True] + [False] * 8)
    v_call = _tc_call(
        _v_body, NVP, BQ,
        [(NVP, 2 * D), (NVP, D), (NVP, D), v_W0a.shape, v_W0b.shape,
         v_W0c.shape, vb0.shape, v_W1.shape, vb1.shape, v_W2.shape, vb2.shape],
        [(NVP, 2 * D)],
        [True, True, True] + [False] * 8)

    gather256 = _make_gather(2 * D)
    gather128 = _make_gather(D)
    seg_c = _make_segsum_tc(2 * D, NB_C, CMAX_C)
    seg_v = _make_segsum_tc(D, NB_V, CMAX_V)

    for r in range(ROUNDS):
        tp, tn = q_call(v_emb, noises[r], q_W0a, q_W0b, qb0,
                        q_W1, qb1, q_W2, qb2)
        gp = gather256(tp, pg)
        gn = gather256(tn, ng)
        accp = seg_c(pgb, pob, pdstr, gp)
        accn = seg_c(ngb, nob, ndstr, gn)
        gcp = gather128(c, pg2)
        gcn = gather128(c, ng2)
        cp = seg_v(pgb2, pob2, pdstr2, gcp)
        cn = seg_v(ngb2, nob2, ndstr2, gcn)
        c = c_call(accp, accn, c, c_W0a, c_W0b, c_W0c, cb0,
                   c_W1, cb1, c_W2, cb2)
        v_emb = v_call(v_emb, cp, cn, v_W0a, v_W0b, v_W0c,
                       vb0, v_W1, vb1, v_W2, vb2)

    return v_emb[:N_V, :D], v_emb[:N_V, D:], c[:N_C]
